# Initial kernel scaffold; baseline (speedup 1.0000x reference)
import jax
import jax.numpy as jnp
from jax.experimental import pallas as pl


def kernel(x, edge_index, W, b):
    raise NotImplementedError("write your pallas kernel here")


# speedup vs baseline: 3.4840x; 3.4840x over previous
import functools

import jax
import jax.numpy as jnp
from jax import lax
from jax.experimental import pallas as pl
from jax.experimental.pallas import tpu as pltpu
from jax.experimental.pallas import tpu_sc as plsc

N_NODES = 10000
N_EDGES = 320000
D_IN = 128
D_OUT = 128

NC = 2
NS = 16
NW = NC * NS
CHUNK = 128
K = 80
E_PAD = NW * CHUNK * K
NPAD = 10240
RPT = NPAD // NS

_mesh = plsc.VectorSubcoreMesh(core_axis_name="c", subcore_axis_name="s")


@functools.partial(
    pl.kernel,
    out_type=(
        jax.ShapeDtypeStruct((NC, NPAD, D_IN), jnp.float32),
        jax.ShapeDtypeStruct((NW, NPAD), jnp.float32),
    ),
    mesh=_mesh,
    compiler_params=pltpu.CompilerParams(needs_layout_passes=False),
    scratch_types=[
        pltpu.VMEM((K, CHUNK), jnp.int32),
        pltpu.VMEM((K, CHUNK), jnp.int32),
        pltpu.VMEM((CHUNK, D_IN), jnp.float32),
        pltpu.VMEM((NPAD,), jnp.float32),
        pltpu.VMEM_SHARED((NPAD, D_IN), jnp.float32),
        pltpu.SemaphoreType.DMA,
    ],
)
def _sc_aggregate(x_hbm, src_hbm, dst_hbm, zeros_hbm, zrow_hbm,
                  acc_hbm, deg_hbm,
                  srcv, dstv, rows, degv, acc_s, sem):
    c = lax.axis_index("c")
    s = lax.axis_index("s")
    wid = s * NC + c

    pltpu.sync_copy(zeros_hbm, acc_s.at[pl.ds(s * RPT, RPT)])
    pltpu.sync_copy(zrow_hbm, degv)
    pltpu.sync_copy(src_hbm.at[wid], srcv)
    pltpu.sync_copy(dst_hbm.at[wid], dstv)
    plsc.subcore_barrier()

    ones16 = jnp.full((16,), 1.0, dtype=jnp.float32)

    def body(j, carry):
        pltpu.async_copy(x_hbm.at[srcv.at[j]], rows, sem).wait()
        pltpu.sync_copy(rows, acc_s.at[dstv.at[j]], add=True)
        for g in range(CHUNK // 16):
            idx = dstv.at[j][pl.ds(g * 16, 16)]
            plsc.addupdate_scatter(degv, [idx], ones16)
        return carry

    lax.fori_loop(0, K, body, 0)
    plsc.subcore_barrier()

    pltpu.sync_copy(acc_s.at[pl.ds(s * RPT, RPT)],
                    acc_hbm.at[c, pl.ds(s * RPT, RPT)])
    pltpu.sync_copy(degv, deg_hbm.at[wid])


def _tc_finish_body(acc0_ref, acc1_ref, deg_ref, w_ref, b_ref, out_ref):
    acc = acc0_ref[...] + acc1_ref[...]
    deg = jnp.sum(deg_ref[...], axis=1, keepdims=True)
    degc = jnp.maximum(deg, 1.0)
    scaled = acc / degc
    out_ref[...] = (
        lax.dot_general(scaled, w_ref[...], (((1,), (1,)), ((), ())),
                        preferred_element_type=jnp.float32)
        + jnp.where(deg > 0.0, 1.0, 0.0) * b_ref[...]
    )


def kernel(x, edge_index, W, b):
    src = edge_index[0]
    dst = edge_index[1]
    pad = E_PAD - N_EDGES
    src_p = jnp.concatenate([src, jnp.zeros((pad,), jnp.int32)])
    dst_p = jnp.concatenate([dst, jnp.full((pad,), N_NODES, jnp.int32)])
    src3 = src_p.reshape(NW, K, CHUNK)
    dst3 = dst_p.reshape(NW, K, CHUNK)
    zeros2d = jnp.zeros((RPT, D_IN), jnp.float32)
    zrow = jnp.zeros((NPAD,), jnp.float32)

    acc2, deg2 = _sc_aggregate(x, src3, dst3, zeros2d, zrow)

    acc0 = acc2[0, :N_NODES, :]
    acc1 = acc2[1, :N_NODES, :]
    degm = deg2.T[:N_NODES, :]

    blk = 2000
    grid = N_NODES // blk
    out = pl.pallas_call(
        _tc_finish_body,
        grid=(grid,),
        in_specs=[
            pl.BlockSpec((blk, D_IN), lambda i: (i, 0)),
            pl.BlockSpec((blk, D_IN), lambda i: (i, 0)),
            pl.BlockSpec((blk, NW), lambda i: (i, 0)),
            pl.BlockSpec((D_OUT, D_IN), lambda i: (0, 0)),
            pl.BlockSpec((1, D_OUT), lambda i: (0, 0)),
        ],
        out_specs=pl.BlockSpec((blk, D_OUT), lambda i: (i, 0)),
        out_shape=jax.ShapeDtypeStruct((N_NODES, D_OUT), jnp.float32),
    )(acc0, acc1, degm, W, b.reshape(1, D_OUT))
    return out


# speedup vs baseline: 3.8913x; 1.1169x over previous
import functools

import jax
import jax.numpy as jnp
from jax import lax
from jax.experimental import pallas as pl
from jax.experimental.pallas import tpu as pltpu
from jax.experimental.pallas import tpu_sc as plsc

N_NODES = 10000
N_EDGES = 320000
D_IN = 128
D_OUT = 128

NC = 2
NS = 16
NW = NC * NS
CHUNK = 128
K = 80
G = 8
NG = K // G
E_PAD = NW * CHUNK * K
NPAD = 10240
RPT = NPAD // NS

_mesh = plsc.VectorSubcoreMesh(core_axis_name="c", subcore_axis_name="s")


@functools.partial(
    pl.kernel,
    out_type=(
        jax.ShapeDtypeStruct((NC, NPAD, D_IN), jnp.float32),
        jax.ShapeDtypeStruct((NW, NPAD), jnp.float32),
    ),
    mesh=_mesh,
    compiler_params=pltpu.CompilerParams(needs_layout_passes=False),
    scratch_types=[
        pltpu.VMEM((2, G, CHUNK), jnp.int32),
        pltpu.VMEM((2, G, CHUNK), jnp.int32),
        pltpu.VMEM((CHUNK, D_IN), jnp.float32),
        pltpu.VMEM((CHUNK, D_IN), jnp.float32),
        pltpu.VMEM((NPAD,), jnp.float32),
        pltpu.VMEM_SHARED((NPAD, D_IN), jnp.float32),
        pltpu.SemaphoreType.DMA,
        pltpu.SemaphoreType.DMA,
        pltpu.SemaphoreType.DMA,
    ],
)
def _sc_aggregate(x_hbm, src_hbm, dst_hbm, zeros_hbm, zrow_hbm,
                  acc_hbm, deg_hbm,
                  srcv, dstv, rows_a, rows_b, degv, acc_s,
                  sem_a, sem_b, sem_i):
    c = lax.axis_index("c")
    s = lax.axis_index("s")
    wid = s * NC + c

    pltpu.sync_copy(zeros_hbm, acc_s.at[pl.ds(s * RPT, RPT)])
    pltpu.sync_copy(zrow_hbm, degv)
    pltpu.sync_copy(src_hbm.at[wid, 0], srcv.at[0])
    pltpu.sync_copy(dst_hbm.at[wid, 0], dstv.at[0])
    plsc.subcore_barrier()

    ones16 = jnp.full((16,), 1.0, dtype=jnp.float32)

    def deg_update(d_cur, j):
        for q in range(CHUNK // 16):
            idx = d_cur.at[j][pl.ds(q * 16, 16)]
            plsc.addupdate_scatter(degv, [idx], ones16)

    def wait_gather(rows, sem):
        pltpu.make_async_copy(x_hbm.at[pl.ds(0, CHUNK)], rows, sem).wait()

    pltpu.async_copy(x_hbm.at[srcv.at[0, 0]], rows_a, sem_a)

    for g in range(NG):
        s_cur = srcv.at[g % 2]
        d_cur = dstv.at[g % 2]
        if g + 1 < NG:
            pltpu.async_copy(src_hbm.at[wid, g + 1], srcv.at[(g + 1) % 2],
                             sem_i)
            pltpu.async_copy(dst_hbm.at[wid, g + 1], dstv.at[(g + 1) % 2],
                             sem_i)

        def body(i, carry, g=g, s_cur=s_cur, d_cur=d_cur):
            j0 = 2 * i
            j1 = 2 * i + 1
            pltpu.async_copy(x_hbm.at[s_cur.at[j1]], rows_b, sem_b)
            deg_update(d_cur, j0)
            wait_gather(rows_a, sem_a)
            pltpu.sync_copy(rows_a, acc_s.at[d_cur.at[j0]], add=True)

            @pl.when(i + 1 < G // 2)
            def _():
                pltpu.async_copy(x_hbm.at[s_cur.at[j0 + 2]], rows_a, sem_a)

            if g + 1 < NG:
                @pl.when(i + 1 == G // 2)
                def _():
                    pltpu.make_async_copy(
                        src_hbm.at[wid, g + 1], srcv.at[(g + 1) % 2],
                        sem_i).wait()
                    pltpu.make_async_copy(
                        dst_hbm.at[wid, g + 1], dstv.at[(g + 1) % 2],
                        sem_i).wait()
                    pltpu.async_copy(x_hbm.at[srcv.at[(g + 1) % 2, 0]],
                                     rows_a, sem_a)

            deg_update(d_cur, j1)
            wait_gather(rows_b, sem_b)
            pltpu.sync_copy(rows_b, acc_s.at[d_cur.at[j1]], add=True)
            return carry

        lax.fori_loop(0, G // 2, body, 0)

    plsc.subcore_barrier()

    pltpu.sync_copy(acc_s.at[pl.ds(s * RPT, RPT)],
                    acc_hbm.at[c, pl.ds(s * RPT, RPT)])
    pltpu.sync_copy(degv, deg_hbm.at[wid])


def _tc_finish_body(acc0_ref, acc1_ref, deg_ref, w_ref, b_ref, out_ref):
    acc = acc0_ref[...] + acc1_ref[...]
    deg = jnp.sum(deg_ref[...], axis=1, keepdims=True)
    degc = jnp.maximum(deg, 1.0)
    scaled = acc / degc
    out_ref[...] = (
        lax.dot_general(scaled, w_ref[...], (((1,), (1,)), ((), ())),
                        preferred_element_type=jnp.float32)
        + jnp.where(deg > 0.0, 1.0, 0.0) * b_ref[...]
    )


def kernel(x, edge_index, W, b):
    src = edge_index[0]
    dst = edge_index[1]
    pad = E_PAD - N_EDGES
    src_p = jnp.concatenate([src, jnp.zeros((pad,), jnp.int32)])
    dst_p = jnp.concatenate([dst, jnp.full((pad,), N_NODES, jnp.int32)])
    src3 = src_p.reshape(NW, NG, G, CHUNK)
    dst3 = dst_p.reshape(NW, NG, G, CHUNK)
    zeros2d = jnp.zeros((RPT, D_IN), jnp.float32)
    zrow = jnp.zeros((NPAD,), jnp.float32)

    acc2, deg2 = _sc_aggregate(x, src3, dst3, zeros2d, zrow)

    acc0 = acc2[0, :N_NODES, :]
    acc1 = acc2[1, :N_NODES, :]
    degm = deg2.T[:N_NODES, :]

    blk = 2000
    grid = N_NODES // blk
    out = pl.pallas_call(
        _tc_finish_body,
        grid=(grid,),
        in_specs=[
            pl.BlockSpec((blk, D_IN), lambda i: (i, 0)),
            pl.BlockSpec((blk, D_IN), lambda i: (i, 0)),
            pl.BlockSpec((blk, NW), lambda i: (i, 0)),
            pl.BlockSpec((D_OUT, D_IN), lambda i: (0, 0)),
            pl.BlockSpec((1, D_OUT), lambda i: (0, 0)),
        ],
        out_specs=pl.BlockSpec((blk, D_OUT), lambda i: (i, 0)),
        out_shape=jax.ShapeDtypeStruct((N_NODES, D_OUT), jnp.float32),
    )(acc0, acc1, degm, W, b.reshape(1, D_OUT))
    return out


# speedup vs baseline: 6.6615x; 1.7119x over previous
import functools

import jax
import jax.numpy as jnp
from jax import lax
from jax.experimental import pallas as pl
from jax.experimental.pallas import tpu as pltpu
from jax.experimental.pallas import tpu_sc as plsc

N_NODES = 10000
N_EDGES = 320000
D_IN = 128
D_OUT = 128

NC = 2
NS = 16
NW = NC * NS
NHALF = N_NODES // 2

CHUNK = 32
GEDGE = 128
NGRP = 160
E_PAD = NS * NGRP * GEDGE

XROWS = 10112
AROWS = 5056
TRASH = NHALF
NPAD = 10240
KD = 80

_mesh = plsc.VectorSubcoreMesh(core_axis_name="c", subcore_axis_name="s")


@functools.partial(
    pl.kernel,
    out_type=jax.ShapeDtypeStruct((NC, AROWS, D_IN), jnp.float32),
    mesh=_mesh,
    compiler_params=pltpu.CompilerParams(needs_layout_passes=False),
    scratch_types=[
        pltpu.VMEM((2, GEDGE), jnp.int32),
        pltpu.VMEM((2, GEDGE), jnp.int32),
        pltpu.VMEM((4, CHUNK), jnp.int32),
        pltpu.VMEM((CHUNK, D_IN), jnp.float32),
        pltpu.VMEM((CHUNK, D_IN), jnp.float32),
        pltpu.VMEM_SHARED((XROWS, D_IN), jnp.float32),
        pltpu.VMEM_SHARED((AROWS, D_IN), jnp.float32),
        pltpu.SemaphoreType.DMA,
        pltpu.SemaphoreType.DMA,
        pltpu.SemaphoreType.DMA,
    ],
)
def _sc_aggregate(x_hbm, src_hbm, dst_hbm, zeros_hbm, acc_hbm,
                  srcv, dstv, dsc, rows_a, rows_b, xs_s, acc_s,
                  sem_a, sem_b, sem_i):
    c = lax.axis_index("c")
    s = lax.axis_index("s")

    pltpu.sync_copy(x_hbm.at[pl.ds(s * (XROWS // NS), XROWS // NS)],
                    xs_s.at[pl.ds(s * (XROWS // NS), XROWS // NS)])

    @pl.when(s < 8)
    def _():
        pltpu.sync_copy(zeros_hbm, acc_s.at[pl.ds(s * (AROWS // 8),
                                                  AROWS // 8)])

    pltpu.sync_copy(src_hbm.at[s, 0], srcv.at[0])
    pltpu.sync_copy(dst_hbm.at[s, 0], dstv.at[0])
    plsc.subcore_barrier()

    iota16 = lax.iota(jnp.int32, 16)
    trash16 = TRASH + (iota16 & 7)
    base = c * NHALF

    def wait_gather(rows, sem):
        pltpu.make_async_copy(xs_s.at[pl.ds(0, CHUNK)], rows, sem).wait()

    def wait_idx(g1, slot):
        pltpu.make_async_copy(src_hbm.at[s, g1], srcv.at[slot], sem_i).wait()
        pltpu.make_async_copy(dst_hbm.at[s, g1], dstv.at[slot], sem_i).wait()

    pltpu.async_copy(xs_s.at[srcv.at[0, pl.ds(0, CHUNK)]], rows_a, sem_a)

    def body(grp, carry):
        p = grp % 2
        pn = 1 - p
        d_cur = dstv.at[p]

        for v in range(GEDGE // 16):
            d16 = d_cur[pl.ds(v * 16, 16)]
            loc = d16 - base
            bad = (loc < 0) | (loc >= NHALF)
            dsc.at[v // 2][pl.ds((v % 2) * 16, 16)] = jnp.where(
                bad, trash16, loc)

        @pl.when(grp + 1 < NGRP)
        def _():
            pltpu.async_copy(src_hbm.at[s, grp + 1], srcv.at[pn], sem_i)
            pltpu.async_copy(dst_hbm.at[s, grp + 1], dstv.at[pn], sem_i)

        pltpu.async_copy(xs_s.at[srcv.at[p, pl.ds(CHUNK, CHUNK)]], rows_b, sem_b)
        wait_gather(rows_a, sem_a)
        pltpu.sync_copy(rows_a, acc_s.at[dsc.at[0]], add=True)

        pltpu.async_copy(xs_s.at[srcv.at[p, pl.ds(2 * CHUNK, CHUNK)]],
                         rows_a, sem_a)
        wait_gather(rows_b, sem_b)
        pltpu.sync_copy(rows_b, acc_s.at[dsc.at[1]], add=True)

        pltpu.async_copy(xs_s.at[srcv.at[p, pl.ds(3 * CHUNK, CHUNK)]],
                         rows_b, sem_b)
        wait_gather(rows_a, sem_a)
        pltpu.sync_copy(rows_a, acc_s.at[dsc.at[2]], add=True)

        @pl.when(grp + 1 < NGRP)
        def _():
            wait_idx(grp + 1, pn)
            pltpu.async_copy(xs_s.at[srcv.at[pn, pl.ds(0, CHUNK)]],
                             rows_a, sem_a)

        wait_gather(rows_b, sem_b)
        pltpu.sync_copy(rows_b, acc_s.at[dsc.at[3]], add=True)
        return carry

    lax.fori_loop(0, NGRP, body, 0)
    plsc.subcore_barrier()

    @pl.when(s < 8)
    def _():
        pltpu.sync_copy(acc_s.at[pl.ds(s * (AROWS // 8), AROWS // 8)],
                        acc_hbm.at[c, pl.ds(s * (AROWS // 8), AROWS // 8)])


@functools.partial(
    pl.kernel,
    out_type=jax.ShapeDtypeStruct((NW, NPAD), jnp.float32),
    mesh=_mesh,
    compiler_params=pltpu.CompilerParams(needs_layout_passes=False),
    scratch_types=[
        pltpu.VMEM((KD, 128), jnp.int32),
        pltpu.VMEM((NPAD,), jnp.float32),
    ],
)
def _sc_degree(dst_hbm, zrow_hbm, deg_hbm, dslab, degv):
    c = lax.axis_index("c")
    s = lax.axis_index("s")
    wid = s * NC + c

    pltpu.sync_copy(zrow_hbm, degv)
    pltpu.sync_copy(dst_hbm.at[wid], dslab)

    ones16 = jnp.full((16,), 1.0, dtype=jnp.float32)

    def body(j, carry):
        for q in range(128 // 16):
            idx = dslab.at[j][pl.ds(q * 16, 16)]
            plsc.addupdate_scatter(degv, [idx], ones16)
        return carry

    lax.fori_loop(0, KD, body, 0)
    pltpu.sync_copy(degv, deg_hbm.at[wid])


def _tc_finish_body(acc_ref, deg_ref, w_ref, b_ref, out_ref):
    deg = jnp.sum(deg_ref[...], axis=1, keepdims=True)
    degc = jnp.maximum(deg, 1.0)
    scaled = acc_ref[...] / degc
    out_ref[...] = (
        lax.dot_general(scaled, w_ref[...], (((1,), (1,)), ((), ())),
                        preferred_element_type=jnp.float32)
        + jnp.where(deg > 0.0, 1.0, 0.0) * b_ref[...]
    )


def kernel(x, edge_index, W, b):
    src = edge_index[0]
    dst = edge_index[1]
    pad = E_PAD - N_EDGES
    src_p = jnp.concatenate([src, jnp.zeros((pad,), jnp.int32)])
    dst_p = jnp.concatenate([dst, jnp.full((pad,), N_NODES, jnp.int32)])
    src3 = src_p.reshape(NS, NGRP, GEDGE)
    dst3 = dst_p.reshape(NS, NGRP, GEDGE)
    dstd = dst_p.reshape(NW, KD, 128)
    xp = jnp.pad(x, ((0, XROWS - N_NODES), (0, 0)))
    zeros2d = jnp.zeros((AROWS // 8, D_IN), jnp.float32)
    zrow = jnp.zeros((NPAD,), jnp.float32)

    acc2 = _sc_aggregate(xp, src3, dst3, zeros2d)
    deg2 = _sc_degree(dstd, zrow)

    accf = jnp.concatenate([acc2[0, :NHALF], acc2[1, :NHALF]], axis=0)
    degm = deg2.T[:N_NODES, :]

    blk = 2000
    grid = N_NODES // blk
    out = pl.pallas_call(
        _tc_finish_body,
        grid=(grid,),
        in_specs=[
            pl.BlockSpec((blk, D_IN), lambda i: (i, 0)),
            pl.BlockSpec((blk, NW), lambda i: (i, 0)),
            pl.BlockSpec((D_OUT, D_IN), lambda i: (0, 0)),
            pl.BlockSpec((1, D_OUT), lambda i: (0, 0)),
        ],
        out_specs=pl.BlockSpec((blk, D_OUT), lambda i: (i, 0)),
        out_shape=jax.ShapeDtypeStruct((N_NODES, D_OUT), jnp.float32),
    )(accf, degm, W, b.reshape(1, D_OUT))
    return out
